# Initial kernel scaffold; baseline (speedup 1.0000x reference)
import jax
import jax.numpy as jnp
from jax.experimental import pallas as pl


def kernel(features, edge_index, W):
    raise NotImplementedError("write your pallas kernel here")


# speedup vs baseline: 6.3293x; 6.3293x over previous
import functools

import jax
import jax.numpy as jnp
from jax import lax
from jax.experimental import pallas as pl
from jax.experimental.pallas import tpu as pltpu
from jax.experimental.pallas import tpu_sc as plsc

N_NODES = 10000
D_FEAT = 128
N_EDGES = 320000

NC = 2
NS = 16
HALF = N_NODES // NC
EPT = N_EDGES // NS
NVEC = EPT // 16
SEL = EPT + 128
ACC_R = 5120
TRASH = HALF + 8
K = 128
DR = 312

_mesh = plsc.VectorSubcoreMesh(core_axis_name="c", subcore_axis_name="s")


@functools.partial(
    pl.kernel,
    mesh=_mesh,
    out_type=jax.ShapeDtypeStruct((N_NODES, D_FEAT), jnp.float32),
    scratch_types=[
        pltpu.VMEM_SHARED((ACC_R, D_FEAT), jnp.float32),
        pltpu.VMEM((SEL,), jnp.int32),
        pltpu.VMEM((SEL,), jnp.int32),
        pltpu.VMEM((K, D_FEAT), jnp.float32),
        pltpu.VMEM((K,), jnp.int32),
        pltpu.VMEM((K,), jnp.int32),
        pltpu.VMEM((D_FEAT,), jnp.float32),
        pltpu.SemaphoreType.DMA,
    ],
    compiler_params=pltpu.CompilerParams(needs_layout_passes=False),
)
def _gcn_sc(feat, srcv, dstv, w, out, acc, src_sl, dst_sl, rows, gsrc, gdst,
            wv, gsem):
    c = lax.axis_index("c")
    s = lax.axis_index("s")
    zero16 = jnp.zeros((16,), jnp.float32)

    for r in range(64):
        for j in range(8):
            rows[r, pl.ds(j * 16, 16)] = zero16
    for kk in range(5):
        pltpu.sync_copy(rows.at[pl.ds(0, 64)],
                        acc.at[pl.ds(s * 320 + kk * 64, 64)])

    pltpu.sync_copy(srcv.at[pl.ds(s * EPT, EPT)], src_sl.at[pl.ds(0, EPT)])
    pltpu.sync_copy(dstv.at[pl.ds(s * EPT, EPT)], dst_sl.at[pl.ds(0, EPT)])
    pltpu.sync_copy(w, wv)

    plsc.subcore_barrier()

    base_node = c * HALF

    def fbody(i, off):
        sl = src_sl[pl.ds(i * 16, 16)]
        dl = dst_sl[pl.ds(i * 16, 16)] - base_node
        m = (dl >= 0) & (dl < HALF)
        mi = m.astype(jnp.int32)
        pos = jnp.cumsum(mi) - 1 + off
        plsc.store_scatter(src_sl, [pos], sl, mask=m)
        plsc.store_scatter(dst_sl, [pos], dl, mask=m)
        return off + jnp.sum(mi)

    n_sel = lax.fori_loop(0, NVEC, fbody, jnp.int32(0))

    lanes = lax.iota(jnp.int32, 16)
    for i in range(8):
        pidx = n_sel + i * 16 + lanes
        plsc.store_scatter(src_sl, [pidx], jnp.zeros((16,), jnp.int32),
                           mask=jnp.full((16,), True))
        plsc.store_scatter(dst_sl, [pidx], jnp.full((16,), TRASH, jnp.int32),
                           mask=jnp.full((16,), True))

    nch = (n_sel + (K - 1)) >> 7

    def gbody(j, _):
        b = j * K
        for kk in range(8):
            gsrc[pl.ds(kk * 16, 16)] = src_sl[pl.ds(b + kk * 16, 16)]
            gdst[pl.ds(kk * 16, 16)] = dst_sl[pl.ds(b + kk * 16, 16)]
        pltpu.async_copy(feat.at[gsrc], rows, gsem).wait()
        pltpu.sync_copy(rows, acc.at[gdst], add=True)
        return 0

    lax.fori_loop(0, nch, gbody, 0)

    plsc.subcore_barrier()

    wregs = [wv[pl.ds(j * 16, 16)] for j in range(8)]

    def scale_rows(nr):
        def mbody(r, _):
            for j in range(8):
                rows[r, pl.ds(j * 16, 16)] = rows[r, pl.ds(j * 16, 16)] * wregs[j]
            return 0
        lax.fori_loop(0, nr, mbody, 0)

    r0 = s * DR
    for kk in range(3):
        rs = r0 + kk * 104
        pltpu.sync_copy(acc.at[pl.ds(rs, 104)], rows.at[pl.ds(0, 104)])
        scale_rows(104)
        pltpu.sync_copy(rows.at[pl.ds(0, 104)],
                        out.at[pl.ds(c * HALF + rs, 104)])

    @pl.when(s == NS - 1)
    def _tail():
        pltpu.sync_copy(acc.at[pl.ds(NS * DR, 8)], rows.at[pl.ds(0, 8)])
        scale_rows(8)
        pltpu.sync_copy(rows.at[pl.ds(0, 8)],
                        out.at[pl.ds(c * HALF + NS * DR, 8)])


def kernel(features, edge_index, W):
    src = edge_index[0]
    dst = edge_index[1]
    return _gcn_sc(features, src, dst, W)
